# Initial kernel scaffold; baseline (speedup 1.0000x reference)
import jax
import jax.numpy as jnp
from jax.experimental import pallas as pl


def kernel(x, edge_index, edge_attr):
    raise NotImplementedError("write your pallas kernel here")


# speedup vs baseline: 3.6877x; 3.6877x over previous
import functools

import jax
import jax.numpy as jnp
from jax import lax
from jax.experimental import pallas as pl
from jax.experimental.pallas import tpu as pltpu
from jax.experimental.pallas import tpu_sc as plsc

DELTA_X = 0.01
INV_DX2 = 1.0 / (DELTA_X * DELTA_X)

NC = 2
NS = 16
NW = NC * NS


def _sc_scatter_sum(dst_idx, edge_attr, zeros):
    n_edges = dst_idx.shape[0]
    n_nodes, d = zeros.shape
    edges_per_w = n_edges // NW
    chunk = 80
    n_chunks = edges_per_w // chunk
    assert n_chunks * chunk == edges_per_w
    stripe = (n_nodes // NS) // 8 * 8
    tail_off = NS * stripe
    tail = n_nodes - tail_off

    mesh = plsc.VectorSubcoreMesh(core_axis_name="c", subcore_axis_name="s")

    @functools.partial(
        pl.kernel,
        out_type=jax.ShapeDtypeStruct((NC, n_nodes, d), jnp.float32),
        mesh=mesh,
        scratch_types=[
            pltpu.VMEM_SHARED((n_nodes, d), jnp.float32),
            pltpu.VMEM((chunk,), jnp.int32),
            pltpu.VMEM((chunk, d), jnp.float32),
        ],
    )
    def k(idx_hbm, attr_hbm, zeros_hbm, out_hbm, acc, idx_v, attr_v):
        cid = lax.axis_index("c")
        sid = lax.axis_index("s")
        wid = sid * NC + cid

        r0 = sid * stripe
        pltpu.sync_copy(zeros_hbm.at[pl.ds(r0, stripe)],
                        acc.at[pl.ds(r0, stripe)])

        @pl.when(sid == NS - 1)
        def _():
            pltpu.sync_copy(zeros_hbm.at[pl.ds(tail_off, tail)],
                            acc.at[pl.ds(tail_off, tail)])

        plsc.subcore_barrier()

        base = wid * edges_per_w

        @pl.loop(0, n_chunks)
        def _(i):
            off = base + i * chunk
            pltpu.sync_copy(idx_hbm.at[pl.ds(off, chunk)], idx_v)
            pltpu.sync_copy(attr_hbm.at[pl.ds(off, chunk)], attr_v)
            pltpu.sync_copy(attr_v, acc.at[idx_v], add=True)

        plsc.subcore_barrier()
        pltpu.sync_copy(acc.at[pl.ds(r0, stripe)],
                        out_hbm.at[cid, pl.ds(r0, stripe)])

        @pl.when(sid == NS - 1)
        def _():
            pltpu.sync_copy(acc.at[pl.ds(tail_off, tail)],
                            out_hbm.at[cid, pl.ds(tail_off, tail)])

    return k(dst_idx, edge_attr, zeros)


def _combine(partials, x):
    n_nodes, d = x.shape
    blk = 2000
    grid = n_nodes // blk

    def body(p_ref, x_ref, o_ref):
        o_ref[...] = (p_ref[0] + p_ref[1] - 2.0 * x_ref[...]) * INV_DX2

    return pl.pallas_call(
        body,
        grid=(grid,),
        in_specs=[
            pl.BlockSpec((NC, blk, d), lambda i: (0, i, 0)),
            pl.BlockSpec((blk, d), lambda i: (i, 0)),
        ],
        out_specs=pl.BlockSpec((blk, d), lambda i: (i, 0)),
        out_shape=jax.ShapeDtypeStruct((n_nodes, d), jnp.float32),
    )(partials, x)


def kernel(x, edge_index, edge_attr):
    dst = edge_index[1].astype(jnp.int32)
    zeros = jnp.zeros(x.shape, jnp.float32)
    partials = _sc_scatter_sum(dst, edge_attr, zeros)
    return _combine(partials, x)


# speedup vs baseline: 7.9406x; 2.1533x over previous
import functools

import jax
import jax.numpy as jnp
from jax import lax
from jax.experimental import pallas as pl
from jax.experimental.pallas import tpu as pltpu
from jax.experimental.pallas import tpu_sc as plsc

DELTA_X = 0.01
INV_DX2 = 1.0 / (DELTA_X * DELTA_X)

NC = 2
NS = 16
NW = NC * NS


def _sc_scatter_sum(dst_idx, edge_attr, zeros):
    n_edges = dst_idx.shape[0]
    n_nodes, d = zeros.shape
    edges_per_w = n_edges // NW
    chunk = 80
    n_chunks = edges_per_w // chunk
    assert n_chunks * chunk == edges_per_w
    stripe = (n_nodes // NS) // 8 * 8
    tail_off = NS * stripe
    tail = n_nodes - tail_off

    mesh = plsc.VectorSubcoreMesh(core_axis_name="c", subcore_axis_name="s")

    nbuf = 3
    rounds = n_chunks // nbuf
    tail_chunks = n_chunks - rounds * nbuf

    @functools.partial(
        pl.kernel,
        out_type=jax.ShapeDtypeStruct((NC, n_nodes, d), jnp.float32),
        mesh=mesh,
        scratch_types=[
            pltpu.VMEM_SHARED((n_nodes, d), jnp.float32),
            pltpu.VMEM((n_chunks, chunk), jnp.int32),
            pltpu.VMEM((nbuf, chunk, d), jnp.float32),
            pltpu.SemaphoreType.DMA((nbuf,)),
        ],
    )
    def k(idx_hbm, attr_hbm, zeros_hbm, out_hbm, acc, idx_v, attr_v, sem):
        cid = lax.axis_index("c")
        sid = lax.axis_index("s")
        wid = sid * NC + cid

        r0 = sid * stripe
        pltpu.sync_copy(zeros_hbm.at[pl.ds(r0, stripe)],
                        acc.at[pl.ds(r0, stripe)])

        @pl.when(sid == NS - 1)
        def _():
            pltpu.sync_copy(zeros_hbm.at[pl.ds(tail_off, tail)],
                            acc.at[pl.ds(tail_off, tail)])

        plsc.subcore_barrier()

        base = wid * edges_per_w
        pltpu.sync_copy(idx_hbm.at[wid], idx_v)

        def attr_slice(i):
            return attr_hbm.at[pl.ds(base + i * chunk, chunk)]

        for b in range(nbuf):
            pltpu.async_copy(attr_slice(b), attr_v.at[b], sem.at[b])

        @pl.loop(0, rounds)
        def _(r):
            for b in range(nbuf):
                i = r * nbuf + b
                pltpu.make_async_copy(attr_slice(i), attr_v.at[b],
                                      sem.at[b]).wait()
                pltpu.sync_copy(attr_v.at[b], acc.at[idx_v.at[i]], add=True)
                nxt = i + nbuf

                @pl.when(nxt < n_chunks)
                def _():
                    pltpu.async_copy(attr_slice(nxt), attr_v.at[b], sem.at[b])

        for t in range(tail_chunks):
            i = rounds * nbuf + t
            pltpu.make_async_copy(attr_slice(i), attr_v.at[t], sem.at[t]).wait()
            pltpu.sync_copy(attr_v.at[t], acc.at[idx_v.at[i]], add=True)

        plsc.subcore_barrier()
        pltpu.sync_copy(acc.at[pl.ds(r0, stripe)],
                        out_hbm.at[cid, pl.ds(r0, stripe)])

        @pl.when(sid == NS - 1)
        def _():
            pltpu.sync_copy(acc.at[pl.ds(tail_off, tail)],
                            out_hbm.at[cid, pl.ds(tail_off, tail)])

    return k(dst_idx.reshape(NW, n_chunks, chunk), edge_attr, zeros)


def _combine(partials, x):
    n_nodes, d = x.shape
    blk = 2000
    grid = n_nodes // blk

    def body(p_ref, x_ref, o_ref):
        o_ref[...] = (p_ref[0] + p_ref[1] - 2.0 * x_ref[...]) * INV_DX2

    return pl.pallas_call(
        body,
        grid=(grid,),
        in_specs=[
            pl.BlockSpec((NC, blk, d), lambda i: (0, i, 0)),
            pl.BlockSpec((blk, d), lambda i: (i, 0)),
        ],
        out_specs=pl.BlockSpec((blk, d), lambda i: (i, 0)),
        out_shape=jax.ShapeDtypeStruct((n_nodes, d), jnp.float32),
    )(partials, x)


def kernel(x, edge_index, edge_attr):
    dst = edge_index[1].astype(jnp.int32)
    zeros = jnp.zeros(x.shape, jnp.float32)
    partials = _sc_scatter_sum(dst, edge_attr, zeros)
    return _combine(partials, x)
